# Initial kernel scaffold; baseline (speedup 1.0000x reference)
import jax
import jax.numpy as jnp
from jax.experimental import pallas as pl


def kernel(query_layer, key_layer, value_layer, attention_mask, local_idx, global_idx, global_key, global_value, global_mask):
    raise NotImplementedError("write your pallas kernel here")


# speedup vs baseline: 3.8920x; 3.8920x over previous
import functools

import jax
import jax.numpy as jnp
import numpy as np
from jax import lax
from jax.experimental import pallas as pl
from jax.experimental.pallas import tpu as pltpu
from jax.experimental.pallas import tpu_sc as plsc

N, H, T, D = 2, 16, 2048, 128
NH = N * H
NBL = 32
TQ = T // NBL
WL = 3 * TQ
HALO_L = TQ
LEXT = T + 2 * HALO_L
TG = 512
WG = 96
HALO_G = 32
GEXT = TG + 2 * HALO_G
NG = 64
KTOT = WL + WG + NG
SCALE = 1.0 / float(np.sqrt(D))

NW = 32
LPW = NH * LEXT // NW
GPW = NH * GEXT // NW
CL = 272
CG = 288
BUF_ROWS = max(CL, CG)


def _sc_gather_body(k2_hbm, v2_hbm, lidx_hbm, gidx_hbm,
                    klo, vlo, kgo, vgo,
                    lidx_v, gidx_v, bufs, gsems, osems):
    nc = 2
    wid = lax.axis_index("s") * nc + lax.axis_index("c")
    base_l = wid * LPW
    base_g = wid * GPW
    pltpu.sync_copy(lidx_hbm.at[pl.ds(base_l, LPW)], lidx_v)
    pltpu.sync_copy(gidx_hbm.at[pl.ds(base_g, GPW)], gidx_v)

    transfers = []
    for c in range(LPW // CL):
        isl = lidx_v.at[pl.ds(c * CL, CL)]
        transfers.append((k2_hbm, isl, klo, base_l + c * CL, CL))
        transfers.append((v2_hbm, isl, vlo, base_l + c * CL, CL))
    for c in range(GPW // CG):
        isl = gidx_v.at[pl.ds(c * CG, CG)]
        transfers.append((k2_hbm, isl, kgo, base_g + c * CG, CG))
        transfers.append((v2_hbm, isl, vgo, base_g + c * CG, CG))

    nt = len(transfers)
    g_handles = [None] * nt
    o_handles = [None] * nt
    for i in range(nt + 1):
        if i < nt:
            b = i % 2
            if i >= 2:
                o_handles[i - 2].wait()
            tbl, isl, _, _, rows = transfers[i]
            g_handles[i] = pltpu.async_copy(tbl.at[isl], bufs[b].at[pl.ds(0, rows)],
                                            gsems[b])
        if i >= 1:
            j = i - 1
            b = j % 2
            g_handles[j].wait()
            _, _, out, off, rows = transfers[j]
            o_handles[j] = pltpu.async_copy(bufs[b].at[pl.ds(0, rows)],
                                            out.at[pl.ds(off, rows)], osems[b])
    o_handles[nt - 2].wait()
    o_handles[nt - 1].wait()


def _sc_gather(k2, v2, lflat, gflat):
    mesh = plsc.VectorSubcoreMesh(core_axis_name="c", subcore_axis_name="s")
    f32 = jnp.float32

    def body(k2_hbm, v2_hbm, lidx_hbm, gidx_hbm, klo, vlo, kgo, vgo,
             lidx_v, gidx_v, buf0, buf1, gs0, gs1, os0, os1):
        _sc_gather_body(k2_hbm, v2_hbm, lidx_hbm, gidx_hbm, klo, vlo, kgo, vgo,
                        lidx_v, gidx_v, (buf0, buf1), (gs0, gs1), (os0, os1))

    return pl.kernel(
        body,
        out_type=(jax.ShapeDtypeStruct((NH * LEXT, D), f32),
                  jax.ShapeDtypeStruct((NH * LEXT, D), f32),
                  jax.ShapeDtypeStruct((NH * GEXT, D), f32),
                  jax.ShapeDtypeStruct((NH * GEXT, D), f32)),
        mesh=mesh,
        scratch_types=[pltpu.VMEM((LPW,), jnp.int32),
                       pltpu.VMEM((GPW,), jnp.int32),
                       pltpu.VMEM((BUF_ROWS, D), f32),
                       pltpu.VMEM((BUF_ROWS, D), f32),
                       pltpu.SemaphoreType.DMA, pltpu.SemaphoreType.DMA,
                       pltpu.SemaphoreType.DMA, pltpu.SemaphoreType.DMA],
    )(k2, v2, lflat, gflat)


def _attn_body(q_ref, kl_ref, vl_ref, kg_ref, vg_ref, gk_ref, gv_ref, m_ref,
               o_ref):
    j = pl.program_id(1)
    g32 = (j // 2) * 32
    q = q_ref[0] * SCALE
    kl = kl_ref[0, pl.ds(j * TQ, WL), :]
    kg = kg_ref[0, pl.ds(g32, WG), :]
    gk = gk_ref[0]
    dn = (((1,), (1,)), ((), ()))
    sl = lax.dot_general(q, kl, dn, preferred_element_type=jnp.float32)
    sg = lax.dot_general(q, kg, dn, preferred_element_type=jnp.float32)
    st = lax.dot_general(q, gk, dn, preferred_element_type=jnp.float32)
    s = jnp.concatenate([sl, sg, st], axis=1) + m_ref[0, 0]
    s = s - jnp.max(s, axis=1, keepdims=True)
    e = jnp.exp(s)
    p = e / jnp.sum(e, axis=1, keepdims=True)
    vl = vl_ref[0, pl.ds(j * TQ, WL), :]
    vg = vg_ref[0, pl.ds(g32, WG), :]
    gv = gv_ref[0]
    dn2 = (((1,), (0,)), ((), ()))
    acc = lax.dot_general(p[:, :WL], vl, dn2, preferred_element_type=jnp.float32)
    acc += lax.dot_general(p[:, WL:WL + WG], vg, dn2,
                           preferred_element_type=jnp.float32)
    acc += lax.dot_general(p[:, WL + WG:], gv, dn2,
                           preferred_element_type=jnp.float32)
    o_ref[0] = acc


def _attn_tc(q3, kl3, vl3, kg3, vg3, gk3, gv3, m4):
    grid = (NH, NBL)
    return pl.pallas_call(
        _attn_body,
        grid=grid,
        in_specs=[
            pl.BlockSpec((1, TQ, D), lambda i, j: (i, j, 0)),
            pl.BlockSpec((1, LEXT, D), lambda i, j: (i, 0, 0)),
            pl.BlockSpec((1, LEXT, D), lambda i, j: (i, 0, 0)),
            pl.BlockSpec((1, GEXT, D), lambda i, j: (i, 0, 0)),
            pl.BlockSpec((1, GEXT, D), lambda i, j: (i, 0, 0)),
            pl.BlockSpec((1, NG, D), lambda i, j: (i, 0, 0)),
            pl.BlockSpec((1, NG, D), lambda i, j: (i, 0, 0)),
            pl.BlockSpec((1, 1, 1, KTOT), lambda i, j: (i, j, 0, 0)),
        ],
        out_specs=pl.BlockSpec((1, TQ, D), lambda i, j: (i, j, 0)),
        out_shape=jax.ShapeDtypeStruct((NH, T, D), jnp.float32),
    )(q3, kl3, vl3, kg3, vg3, gk3, gv3, m4)


def kernel(query_layer, key_layer, value_layer, attention_mask, local_idx,
           global_idx, global_key, global_value, global_mask):
    i32 = jnp.int32
    li = local_idx[..., 0].astype(i32)
    gi = global_idx[..., 0].astype(i32)
    li_ext = jnp.concatenate([li[..., -HALO_L:], li, li[..., :HALO_L]], axis=-1)
    gi_ext = jnp.concatenate([gi[..., -HALO_G:], gi, gi[..., :HALO_G]], axis=-1)
    offs = (jnp.arange(NH, dtype=i32) * T).reshape(N, H, 1)
    lflat = (li_ext + offs).reshape(-1)
    gflat = (gi_ext + offs).reshape(-1)

    k2 = key_layer.reshape(NH * T, D)
    v2 = value_layer.reshape(NH * T, D)
    klo, vlo, kgo, vgo = _sc_gather(k2, v2, lflat, gflat)

    amr = jnp.broadcast_to(attention_mask[:, 0, 0, :][:, None, :], (N, H, T))
    ml = jnp.take_along_axis(amr, li_ext, axis=-1).reshape(NH, LEXT)
    mg = jnp.take_along_axis(amr, gi_ext, axis=-1).reshape(NH, GEXT)
    wl_idx = np.arange(NBL)[:, None] * TQ + np.arange(WL)[None, :]
    wg_idx = (np.arange(NBL) // 2)[:, None] * 32 + np.arange(WG)[None, :]
    m4 = jnp.concatenate(
        [ml[:, wl_idx], mg[:, wg_idx],
         jnp.broadcast_to(global_mask.reshape(NH, 1, NG), (NH, NBL, NG))],
        axis=-1).reshape(NH, NBL, 1, KTOT)

    out = _attn_tc(query_layer.reshape(NH, T, D),
                   klo.reshape(NH, LEXT, D), vlo.reshape(NH, LEXT, D),
                   kgo.reshape(NH, GEXT, D), vgo.reshape(NH, GEXT, D),
                   global_key.reshape(NH, NG, D), global_value.reshape(NH, NG, D),
                   m4)
    return out.reshape(N, H, T, D)


# speedup vs baseline: 8.4235x; 2.1643x over previous
import functools

import jax
import jax.numpy as jnp
import numpy as np
from jax import lax
from jax.experimental import pallas as pl
from jax.experimental.pallas import tpu as pltpu
from jax.experimental.pallas import tpu_sc as plsc

N, H, T, D = 2, 16, 2048, 128
NH = N * H
NBL = 32
TQ = T // NBL
WL = 3 * TQ
HALO_L = TQ
LEXT = T + 2 * HALO_L
TG = 512
WG = 96
HALO_G = 32
GEXT = TG + 2 * HALO_G
NG = 64
KTOT = WL + WG + NG
SCALE = 1.0 / float(np.sqrt(D))

NW = 32
LPW = NH * LEXT // NW
GPW = NH * GEXT // NW
CL = 272
CG = 288
BUF_ROWS = max(CL, CG)


def _sc_gather_body(k2_hbm, v2_hbm, lidx_hbm, gidx_hbm,
                    klo, vlo, kgo, vgo,
                    lidx_v, gidx_v, bufs, gsems, osems):
    nc = 2
    wid = lax.axis_index("s") * nc + lax.axis_index("c")
    base_l = wid * LPW
    base_g = wid * GPW
    pltpu.sync_copy(lidx_hbm.at[pl.ds(base_l, LPW)], lidx_v)
    pltpu.sync_copy(gidx_hbm.at[pl.ds(base_g, GPW)], gidx_v)

    transfers = []
    for c in range(LPW // CL):
        isl = lidx_v.at[pl.ds(c * CL, CL)]
        transfers.append((k2_hbm, isl, klo, base_l + c * CL, CL))
        transfers.append((v2_hbm, isl, vlo, base_l + c * CL, CL))
    for c in range(GPW // CG):
        isl = gidx_v.at[pl.ds(c * CG, CG)]
        transfers.append((k2_hbm, isl, kgo, base_g + c * CG, CG))
        transfers.append((v2_hbm, isl, vgo, base_g + c * CG, CG))

    nt = len(transfers)
    g_handles = [None] * nt
    o_handles = [None] * nt
    for i in range(nt + 1):
        if i < nt:
            b = i % 2
            if i >= 2:
                o_handles[i - 2].wait()
            tbl, isl, _, _, rows = transfers[i]
            g_handles[i] = pltpu.async_copy(tbl.at[isl], bufs[b].at[pl.ds(0, rows)],
                                            gsems[b])
        if i >= 1:
            j = i - 1
            b = j % 2
            g_handles[j].wait()
            _, _, out, off, rows = transfers[j]
            o_handles[j] = pltpu.async_copy(bufs[b].at[pl.ds(0, rows)],
                                            out.at[pl.ds(off, rows)], osems[b])
    o_handles[nt - 2].wait()
    o_handles[nt - 1].wait()


def _sc_gather(k2, v2, lflat, gflat):
    mesh = plsc.VectorSubcoreMesh(core_axis_name="c", subcore_axis_name="s")
    f32 = jnp.float32

    def body(k2_hbm, v2_hbm, lidx_hbm, gidx_hbm, klo, vlo, kgo, vgo,
             lidx_v, gidx_v, buf0, buf1, gs0, gs1, os0, os1):
        _sc_gather_body(k2_hbm, v2_hbm, lidx_hbm, gidx_hbm, klo, vlo, kgo, vgo,
                        lidx_v, gidx_v, (buf0, buf1), (gs0, gs1), (os0, os1))

    return pl.kernel(
        body,
        out_type=(jax.ShapeDtypeStruct((NH * LEXT, D), f32),
                  jax.ShapeDtypeStruct((NH * LEXT, D), f32),
                  jax.ShapeDtypeStruct((NH * GEXT, D), f32),
                  jax.ShapeDtypeStruct((NH * GEXT, D), f32)),
        mesh=mesh,
        scratch_types=[pltpu.VMEM((LPW,), jnp.int32),
                       pltpu.VMEM((GPW,), jnp.int32),
                       pltpu.VMEM((BUF_ROWS, D), f32),
                       pltpu.VMEM((BUF_ROWS, D), f32),
                       pltpu.SemaphoreType.DMA, pltpu.SemaphoreType.DMA,
                       pltpu.SemaphoreType.DMA, pltpu.SemaphoreType.DMA],
    )(k2, v2, lflat, gflat)


JB = 8


def _attn_body(q_ref, kl_ref, vl_ref, kg_ref, vg_ref, gk_ref, gv_ref,
               mlw_ref, mgw_ref, gm_ref, o_ref):
    jb = pl.program_id(1)
    gk = gk_ref[0]
    gv = gv_ref[0]
    gm = gm_ref[0]
    dn = (((1,), (1,)), ((), ()))
    dn2 = (((1,), (0,)), ((), ()))
    for u in range(JB):
        j = jb * JB + u
        g32 = (j // 2) * 32
        q = q_ref[0, pl.ds(u * TQ, TQ), :] * SCALE
        kl = kl_ref[0, pl.ds(j * TQ, WL), :]
        kg = kg_ref[0, pl.ds(g32, WG), :]
        sl = lax.dot_general(q, kl, dn, preferred_element_type=jnp.float32)
        sg = lax.dot_general(q, kg, dn, preferred_element_type=jnp.float32)
        st = lax.dot_general(q, gk, dn, preferred_element_type=jnp.float32)
        sl += mlw_ref[0, u]
        sg += mgw_ref[0, u]
        st += gm
        mx = jnp.maximum(
            jnp.maximum(jnp.max(sl, axis=1, keepdims=True),
                        jnp.max(sg, axis=1, keepdims=True)),
            jnp.max(st, axis=1, keepdims=True))
        el = jnp.exp(sl - mx)
        eg = jnp.exp(sg - mx)
        et = jnp.exp(st - mx)
        den = (jnp.sum(el, axis=1, keepdims=True)
               + jnp.sum(eg, axis=1, keepdims=True)
               + jnp.sum(et, axis=1, keepdims=True))
        vl = vl_ref[0, pl.ds(j * TQ, WL), :]
        vg = vg_ref[0, pl.ds(g32, WG), :]
        acc = lax.dot_general(el, vl, dn2, preferred_element_type=jnp.float32)
        acc += lax.dot_general(eg, vg, dn2, preferred_element_type=jnp.float32)
        acc += lax.dot_general(et, gv, dn2, preferred_element_type=jnp.float32)
        o_ref[0, pl.ds(u * TQ, TQ), :] = acc / den


def _attn_tc(q3, kl3, vl3, kg3, vg3, gk3, gv3, mlw, mgw, gm3):
    grid = (NH, NBL // JB)
    return pl.pallas_call(
        _attn_body,
        grid=grid,
        in_specs=[
            pl.BlockSpec((1, JB * TQ, D), lambda i, j: (i, j, 0)),
            pl.BlockSpec((1, LEXT, D), lambda i, j: (i, 0, 0)),
            pl.BlockSpec((1, LEXT, D), lambda i, j: (i, 0, 0)),
            pl.BlockSpec((1, GEXT, D), lambda i, j: (i, 0, 0)),
            pl.BlockSpec((1, GEXT, D), lambda i, j: (i, 0, 0)),
            pl.BlockSpec((1, NG, D), lambda i, j: (i, 0, 0)),
            pl.BlockSpec((1, NG, D), lambda i, j: (i, 0, 0)),
            pl.BlockSpec((1, JB, 1, WL), lambda i, j: (i, j, 0, 0)),
            pl.BlockSpec((1, JB, 1, WG), lambda i, j: (i, j, 0, 0)),
            pl.BlockSpec((1, 1, NG), lambda i, j: (i, 0, 0)),
        ],
        out_specs=pl.BlockSpec((1, JB * TQ, D), lambda i, j: (i, j, 0)),
        out_shape=jax.ShapeDtypeStruct((NH, T, D), jnp.float32),
    )(q3, kl3, vl3, kg3, vg3, gk3, gv3, mlw, mgw, gm3)


def kernel(query_layer, key_layer, value_layer, attention_mask, local_idx,
           global_idx, global_key, global_value, global_mask):
    i32 = jnp.int32
    li = local_idx[..., 0].astype(i32)
    gi = global_idx[..., 0].astype(i32)
    li_ext = jnp.concatenate([li[..., -HALO_L:], li, li[..., :HALO_L]], axis=-1)
    gi_ext = jnp.concatenate([gi[..., -HALO_G:], gi, gi[..., :HALO_G]], axis=-1)
    offs = (jnp.arange(NH, dtype=i32) * T).reshape(N, H, 1)
    lflat = (li_ext + offs).reshape(-1)
    gflat = (gi_ext + offs).reshape(-1)

    k2 = key_layer.reshape(NH * T, D)
    v2 = value_layer.reshape(NH * T, D)
    klo, vlo, kgo, vgo = _sc_gather(k2, v2, lflat, gflat)

    amr = jnp.broadcast_to(attention_mask[:, 0, 0, :][:, None, :], (N, H, T))
    ml = jnp.take_along_axis(amr, li_ext, axis=-1).reshape(NH, LEXT)
    mg = jnp.take_along_axis(amr, gi_ext, axis=-1).reshape(NH, GEXT)
    wl_idx = np.arange(NBL)[:, None] * TQ + np.arange(WL)[None, :]
    wg_idx = (np.arange(NBL) // 2)[:, None] * 32 + np.arange(WG)[None, :]
    mlw = ml[:, wl_idx].reshape(NH, NBL, 1, WL)
    mgw = mg[:, wg_idx].reshape(NH, NBL, 1, WG)

    out = _attn_tc(query_layer.reshape(NH, T, D),
                   klo.reshape(NH, LEXT, D), vlo.reshape(NH, LEXT, D),
                   kgo.reshape(NH, GEXT, D), vgo.reshape(NH, GEXT, D),
                   global_key.reshape(NH, NG, D), global_value.reshape(NH, NG, D),
                   mlw, mgw, global_mask.reshape(NH, 1, NG))
    return out.reshape(N, H, T, D)


# speedup vs baseline: 11.0983x; 1.3175x over previous
import functools

import jax
import jax.numpy as jnp
import numpy as np
from jax import lax
from jax.experimental import pallas as pl
from jax.experimental.pallas import tpu as pltpu
from jax.experimental.pallas import tpu_sc as plsc

N, H, T, D = 2, 16, 2048, 128
NH = N * H
NBL = 32
TQ = T // NBL
WL = 3 * TQ
HALO_L = TQ
LEXT = T + 2 * HALO_L
TG = 512
WG = 96
HALO_G = 32
GEXT = TG + 2 * HALO_G
NG = 64
KTOT = WL + WG + NG
SCALE = 1.0 / float(np.sqrt(D))

NW = 32
LPW = NH * LEXT // NW
GPW = NH * GEXT // NW
CL = 272
CG = 288
BUF_ROWS = max(CL, CG)


def _sc_gather_body(k2_hbm, v2_hbm, lidx_hbm, gidx_hbm,
                    klo, vlo, kgo, vgo,
                    lidx_v, gidx_v, bufs, gsems, osems):
    nc = 2
    wid = lax.axis_index("s") * nc + lax.axis_index("c")
    base_l = wid * LPW
    base_g = wid * GPW
    pltpu.sync_copy(lidx_hbm.at[pl.ds(base_l, LPW)], lidx_v)
    pltpu.sync_copy(gidx_hbm.at[pl.ds(base_g, GPW)], gidx_v)

    transfers = []
    for c in range(LPW // CL):
        isl = lidx_v.at[pl.ds(c * CL, CL)]
        transfers.append((k2_hbm, isl, klo, base_l + c * CL, CL))
        transfers.append((v2_hbm, isl, vlo, base_l + c * CL, CL))
    for c in range(GPW // CG):
        isl = gidx_v.at[pl.ds(c * CG, CG)]
        transfers.append((k2_hbm, isl, kgo, base_g + c * CG, CG))
        transfers.append((v2_hbm, isl, vgo, base_g + c * CG, CG))

    nt = len(transfers)
    g_handles = [None] * nt
    o_handles = [None] * nt
    for i in range(nt + 1):
        if i < nt:
            b = i % 2
            if i >= 2:
                o_handles[i - 2].wait()
            tbl, isl, _, _, rows = transfers[i]
            g_handles[i] = pltpu.async_copy(tbl.at[isl], bufs[b].at[pl.ds(0, rows)],
                                            gsems[b])
        if i >= 1:
            j = i - 1
            b = j % 2
            g_handles[j].wait()
            _, _, out, off, rows = transfers[j]
            o_handles[j] = pltpu.async_copy(bufs[b].at[pl.ds(0, rows)],
                                            out.at[pl.ds(off, rows)], osems[b])
    o_handles[nt - 2].wait()
    o_handles[nt - 1].wait()


def _sc_gather(k2, v2, lflat, gflat):
    mesh = plsc.VectorSubcoreMesh(core_axis_name="c", subcore_axis_name="s")
    f32 = jnp.float32

    def body(k2_hbm, v2_hbm, lidx_hbm, gidx_hbm, klo, vlo, kgo, vgo,
             lidx_v, gidx_v, buf0, buf1, gs0, gs1, os0, os1):
        _sc_gather_body(k2_hbm, v2_hbm, lidx_hbm, gidx_hbm, klo, vlo, kgo, vgo,
                        lidx_v, gidx_v, (buf0, buf1), (gs0, gs1), (os0, os1))

    return pl.kernel(
        body,
        out_type=(jax.ShapeDtypeStruct((NH * LEXT, D), f32),
                  jax.ShapeDtypeStruct((NH * LEXT, D), f32),
                  jax.ShapeDtypeStruct((NH * GEXT, D), f32),
                  jax.ShapeDtypeStruct((NH * GEXT, D), f32)),
        mesh=mesh,
        scratch_types=[pltpu.VMEM((LPW,), jnp.int32),
                       pltpu.VMEM((GPW,), jnp.int32),
                       pltpu.VMEM((BUF_ROWS, D), f32),
                       pltpu.VMEM((BUF_ROWS, D), f32),
                       pltpu.SemaphoreType.DMA, pltpu.SemaphoreType.DMA,
                       pltpu.SemaphoreType.DMA, pltpu.SemaphoreType.DMA],
    )(k2, v2, lflat, gflat)


JB = 8


def _attn_body(q_ref, kl_ref, vl_ref, kg_ref, vg_ref, gk_ref, gv_ref,
               mlw_ref, mgw_ref, gm_ref, o_ref):
    jb = pl.program_id(1)
    gk = gk_ref[0]
    gv = gv_ref[0]
    gm = gm_ref[0]
    dn = (((1,), (1,)), ((), ()))
    dn2 = (((1,), (0,)), ((), ()))

    def scores(u):
        j = jb * JB + u
        g32 = (j // 2) * 32
        q = q_ref[0, pl.ds(u * TQ, TQ), :] * SCALE
        kl = kl_ref[0, pl.ds(j * TQ, WL), :]
        kg = kg_ref[0, pl.ds(g32, WG), :]
        sl = lax.dot_general(q, kl, dn, preferred_element_type=jnp.float32)
        sg = lax.dot_general(q, kg, dn, preferred_element_type=jnp.float32)
        st = lax.dot_general(q, gk, dn, preferred_element_type=jnp.float32)
        return sl + mlw_ref[0, u], sg + mgw_ref[0, u], st + gm

    def finish(u, sl, sg, st):
        j = jb * JB + u
        g32 = (j // 2) * 32
        mx = jnp.maximum(
            jnp.maximum(jnp.max(sl, axis=1, keepdims=True),
                        jnp.max(sg, axis=1, keepdims=True)),
            jnp.max(st, axis=1, keepdims=True))
        el = jnp.exp(sl - mx)
        eg = jnp.exp(sg - mx)
        et = jnp.exp(st - mx)
        den = (jnp.sum(el, axis=1, keepdims=True)
               + jnp.sum(eg, axis=1, keepdims=True)
               + jnp.sum(et, axis=1, keepdims=True))
        vl = vl_ref[0, pl.ds(j * TQ, WL), :]
        vg = vg_ref[0, pl.ds(g32, WG), :]
        acc = lax.dot_general(el, vl, dn2, preferred_element_type=jnp.float32)
        acc += lax.dot_general(eg, vg, dn2, preferred_element_type=jnp.float32)
        acc += lax.dot_general(et, gv, dn2, preferred_element_type=jnp.float32)
        o_ref[0, pl.ds(u * TQ, TQ), :] = acc / den

    prev = scores(0)
    for u in range(1, JB):
        cur = scores(u)
        finish(u - 1, *prev)
        prev = cur
    finish(JB - 1, *prev)


def _attn_tc(q3, kl3, vl3, kg3, vg3, gk3, gv3, mlw, mgw, gm3):
    grid = (NH, NBL // JB)
    return pl.pallas_call(
        _attn_body,
        grid=grid,
        in_specs=[
            pl.BlockSpec((1, JB * TQ, D), lambda i, j: (i, j, 0)),
            pl.BlockSpec((1, LEXT, D), lambda i, j: (i, 0, 0)),
            pl.BlockSpec((1, LEXT, D), lambda i, j: (i, 0, 0)),
            pl.BlockSpec((1, GEXT, D), lambda i, j: (i, 0, 0)),
            pl.BlockSpec((1, GEXT, D), lambda i, j: (i, 0, 0)),
            pl.BlockSpec((1, NG, D), lambda i, j: (i, 0, 0)),
            pl.BlockSpec((1, NG, D), lambda i, j: (i, 0, 0)),
            pl.BlockSpec((1, JB, 1, WL), lambda i, j: (i, j, 0, 0)),
            pl.BlockSpec((1, JB, 1, WG), lambda i, j: (i, j, 0, 0)),
            pl.BlockSpec((1, 1, NG), lambda i, j: (i, 0, 0)),
        ],
        out_specs=pl.BlockSpec((1, JB * TQ, D), lambda i, j: (i, j, 0)),
        out_shape=jax.ShapeDtypeStruct((NH, T, D), jnp.float32),
    )(q3, kl3, vl3, kg3, vg3, gk3, gv3, mlw, mgw, gm3)


def kernel(query_layer, key_layer, value_layer, attention_mask, local_idx,
           global_idx, global_key, global_value, global_mask):
    i32 = jnp.int32
    li = local_idx[..., 0].astype(i32)
    gi = global_idx[..., 0].astype(i32)
    li_ext = jnp.concatenate([li[..., -HALO_L:], li, li[..., :HALO_L]], axis=-1)
    gi_ext = jnp.concatenate([gi[..., -HALO_G:], gi, gi[..., :HALO_G]], axis=-1)
    offs = (jnp.arange(NH, dtype=i32) * T).reshape(N, H, 1)
    lflat = (li_ext + offs).reshape(-1)
    gflat = (gi_ext + offs).reshape(-1)

    k2 = key_layer.reshape(NH * T, D)
    v2 = value_layer.reshape(NH * T, D)
    klo, vlo, kgo, vgo = _sc_gather(k2, v2, lflat, gflat)

    amr = jnp.broadcast_to(attention_mask[:, 0, 0, :][:, None, :], (N, H, T))
    ml = jnp.take_along_axis(amr, li_ext, axis=-1).reshape(NH, LEXT)
    mg = jnp.take_along_axis(amr, gi_ext, axis=-1).reshape(NH, GEXT)
    wl_idx = np.arange(NBL)[:, None] * TQ + np.arange(WL)[None, :]
    wg_idx = (np.arange(NBL) // 2)[:, None] * 32 + np.arange(WG)[None, :]
    mlw = ml[:, wl_idx].reshape(NH, NBL, 1, WL)
    mgw = mg[:, wg_idx].reshape(NH, NBL, 1, WG)

    out = _attn_tc(query_layer.reshape(NH, T, D),
                   klo.reshape(NH, LEXT, D), vlo.reshape(NH, LEXT, D),
                   kgo.reshape(NH, GEXT, D), vgo.reshape(NH, GEXT, D),
                   global_key.reshape(NH, NG, D), global_value.reshape(NH, NG, D),
                   mlw, mgw, global_mask.reshape(NH, 1, NG))
    return out.reshape(N, H, T, D)


# speedup vs baseline: 15.1802x; 1.3678x over previous
import functools

import jax
import jax.numpy as jnp
import numpy as np
from jax import lax
from jax.experimental import pallas as pl
from jax.experimental.pallas import tpu as pltpu
from jax.experimental.pallas import tpu_sc as plsc

N, H, T, D = 2, 16, 2048, 128
NH = N * H
NBL = 32
TQ = T // NBL
WL = 3 * TQ
HALO_L = TQ
LEXT = T + 2 * HALO_L
TG = 512
WG = 96
HALO_G = 32
GEXT = TG + 2 * HALO_G
NG = 64
KTOT = WL + WG + NG
SCALE = 1.0 / float(np.sqrt(D))

NW = 32
LPW = NH * LEXT // NW
GPW = NH * GEXT // NW
CL = 272
CG = 288
BUF_ROWS = max(CL, CG)


def _sc_gather_body(k2_hbm, v2_hbm, lidx_hbm, gidx_hbm,
                    klo, vlo, kgo, vgo,
                    lidx_v, gidx_v, bufs, gsems, osems):
    nc = 2
    wid = lax.axis_index("s") * nc + lax.axis_index("c")
    base_l = wid * LPW
    base_g = wid * GPW
    pltpu.sync_copy(lidx_hbm.at[pl.ds(base_l, LPW)], lidx_v)
    pltpu.sync_copy(gidx_hbm.at[pl.ds(base_g, GPW)], gidx_v)

    transfers = []
    for c in range(LPW // CL):
        isl = lidx_v.at[pl.ds(c * CL, CL)]
        transfers.append((k2_hbm, isl, klo, base_l + c * CL, CL))
        transfers.append((v2_hbm, isl, vlo, base_l + c * CL, CL))
    for c in range(GPW // CG):
        isl = gidx_v.at[pl.ds(c * CG, CG)]
        transfers.append((k2_hbm, isl, kgo, base_g + c * CG, CG))
        transfers.append((v2_hbm, isl, vgo, base_g + c * CG, CG))

    nt = len(transfers)
    g_handles = [None] * nt
    o_handles = [None] * nt
    for i in range(nt + 1):
        if i < nt:
            b = i % 2
            if i >= 2:
                o_handles[i - 2].wait()
            tbl, isl, _, _, rows = transfers[i]
            g_handles[i] = pltpu.async_copy(tbl.at[isl], bufs[b].at[pl.ds(0, rows)],
                                            gsems[b])
        if i >= 1:
            j = i - 1
            b = j % 2
            g_handles[j].wait()
            _, _, out, off, rows = transfers[j]
            o_handles[j] = pltpu.async_copy(bufs[b].at[pl.ds(0, rows)],
                                            out.at[pl.ds(off, rows)], osems[b])
    o_handles[nt - 2].wait()
    o_handles[nt - 1].wait()


def _sc_gather(k2, v2, lflat, gflat):
    mesh = plsc.VectorSubcoreMesh(core_axis_name="c", subcore_axis_name="s")
    f32 = jnp.float32

    def body(k2_hbm, v2_hbm, lidx_hbm, gidx_hbm, klo, vlo, kgo, vgo,
             lidx_v, gidx_v, buf0, buf1, gs0, gs1, os0, os1):
        _sc_gather_body(k2_hbm, v2_hbm, lidx_hbm, gidx_hbm, klo, vlo, kgo, vgo,
                        lidx_v, gidx_v, (buf0, buf1), (gs0, gs1), (os0, os1))

    return pl.kernel(
        body,
        out_type=(jax.ShapeDtypeStruct((NH * LEXT, D), f32),
                  jax.ShapeDtypeStruct((NH * LEXT, D), f32),
                  jax.ShapeDtypeStruct((NH * GEXT, D), f32),
                  jax.ShapeDtypeStruct((NH * GEXT, D), f32)),
        mesh=mesh,
        scratch_types=[pltpu.VMEM((LPW,), jnp.int32),
                       pltpu.VMEM((GPW,), jnp.int32),
                       pltpu.VMEM((BUF_ROWS, D), f32),
                       pltpu.VMEM((BUF_ROWS, D), f32),
                       pltpu.SemaphoreType.DMA, pltpu.SemaphoreType.DMA,
                       pltpu.SemaphoreType.DMA, pltpu.SemaphoreType.DMA],
    )(k2, v2, lflat, gflat)


JB = 16


def _attn_body(q_ref, kl_ref, vl_ref, kg_ref, vg_ref, gk_ref, gv_ref,
               mlw_ref, mgw_ref, gm_ref, o_ref):
    jb = pl.program_id(1)
    gk = gk_ref[0]
    gv = gv_ref[0]
    gm = gm_ref[0]
    dn = (((1,), (1,)), ((), ()))
    dn2 = (((1,), (0,)), ((), ()))

    def scores(u):
        j = jb * JB + u
        g32 = (j // 2) * 32
        q = q_ref[0, pl.ds(u * TQ, TQ), :] * SCALE
        kl = kl_ref[0, pl.ds(j * TQ, WL), :]
        kg = kg_ref[0, pl.ds(g32, WG), :]
        sl = lax.dot_general(q, kl, dn, preferred_element_type=jnp.float32)
        sg = lax.dot_general(q, kg, dn, preferred_element_type=jnp.float32)
        st = lax.dot_general(q, gk, dn, preferred_element_type=jnp.float32)
        return sl + mlw_ref[0, u], sg + mgw_ref[0, u], st + gm

    def finish(u, sl, sg, st):
        j = jb * JB + u
        g32 = (j // 2) * 32
        mx = jnp.maximum(
            jnp.maximum(jnp.max(sl, axis=1, keepdims=True),
                        jnp.max(sg, axis=1, keepdims=True)),
            jnp.max(st, axis=1, keepdims=True))
        el = jnp.exp(sl - mx)
        eg = jnp.exp(sg - mx)
        et = jnp.exp(st - mx)
        den = (jnp.sum(el, axis=1, keepdims=True)
               + jnp.sum(eg, axis=1, keepdims=True)
               + jnp.sum(et, axis=1, keepdims=True))
        vl = vl_ref[0, pl.ds(j * TQ, WL), :]
        vg = vg_ref[0, pl.ds(g32, WG), :]
        acc = lax.dot_general(el, vl, dn2, preferred_element_type=jnp.float32)
        acc += lax.dot_general(eg, vg, dn2, preferred_element_type=jnp.float32)
        acc += lax.dot_general(et, gv, dn2, preferred_element_type=jnp.float32)
        o_ref[0, pl.ds(u * TQ, TQ), :] = acc / den

    DEPTH = 5
    pipe = [scores(u) for u in range(DEPTH)]
    for u in range(DEPTH, JB):
        pipe.append(scores(u))
        finish(u - DEPTH, *pipe.pop(0))
    for w, s in enumerate(pipe):
        finish(JB - DEPTH + w, *s)


def _attn_tc(q3, kl3, vl3, kg3, vg3, gk3, gv3, mlw, mgw, gm3):
    grid = (NH, NBL // JB)
    return pl.pallas_call(
        _attn_body,
        grid=grid,
        in_specs=[
            pl.BlockSpec((1, JB * TQ, D), lambda i, j: (i, j, 0)),
            pl.BlockSpec((1, LEXT, D), lambda i, j: (i, 0, 0)),
            pl.BlockSpec((1, LEXT, D), lambda i, j: (i, 0, 0)),
            pl.BlockSpec((1, GEXT, D), lambda i, j: (i, 0, 0)),
            pl.BlockSpec((1, GEXT, D), lambda i, j: (i, 0, 0)),
            pl.BlockSpec((1, NG, D), lambda i, j: (i, 0, 0)),
            pl.BlockSpec((1, NG, D), lambda i, j: (i, 0, 0)),
            pl.BlockSpec((1, JB, 1, WL), lambda i, j: (i, j, 0, 0)),
            pl.BlockSpec((1, JB, 1, WG), lambda i, j: (i, j, 0, 0)),
            pl.BlockSpec((1, 1, NG), lambda i, j: (i, 0, 0)),
        ],
        out_specs=pl.BlockSpec((1, JB * TQ, D), lambda i, j: (i, j, 0)),
        out_shape=jax.ShapeDtypeStruct((NH, T, D), jnp.float32),
    )(q3, kl3, vl3, kg3, vg3, gk3, gv3, mlw, mgw, gm3)


def kernel(query_layer, key_layer, value_layer, attention_mask, local_idx,
           global_idx, global_key, global_value, global_mask):
    i32 = jnp.int32
    li = local_idx[..., 0].astype(i32)
    gi = global_idx[..., 0].astype(i32)
    li_ext = jnp.concatenate([li[..., -HALO_L:], li, li[..., :HALO_L]], axis=-1)
    gi_ext = jnp.concatenate([gi[..., -HALO_G:], gi, gi[..., :HALO_G]], axis=-1)
    offs = (jnp.arange(NH, dtype=i32) * T).reshape(N, H, 1)
    lflat = (li_ext + offs).reshape(-1)
    gflat = (gi_ext + offs).reshape(-1)

    k2 = key_layer.reshape(NH * T, D)
    v2 = value_layer.reshape(NH * T, D)
    klo, vlo, kgo, vgo = _sc_gather(k2, v2, lflat, gflat)

    amr = jnp.broadcast_to(attention_mask[:, 0, 0, :][:, None, :], (N, H, T))
    ml = jnp.take_along_axis(amr, li_ext, axis=-1).reshape(NH, LEXT)
    mg = jnp.take_along_axis(amr, gi_ext, axis=-1).reshape(NH, GEXT)
    wl_idx = np.arange(NBL)[:, None] * TQ + np.arange(WL)[None, :]
    wg_idx = (np.arange(NBL) // 2)[:, None] * 32 + np.arange(WG)[None, :]
    mlw = ml[:, wl_idx].reshape(NH, NBL, 1, WL)
    mgw = mg[:, wg_idx].reshape(NH, NBL, 1, WG)

    out = _attn_tc(query_layer.reshape(NH, T, D),
                   klo.reshape(NH, LEXT, D), vlo.reshape(NH, LEXT, D),
                   kgo.reshape(NH, GEXT, D), vgo.reshape(NH, GEXT, D),
                   global_key.reshape(NH, NG, D), global_value.reshape(NH, NG, D),
                   mlw, mgw, global_mask.reshape(NH, 1, NG))
    return out.reshape(N, H, T, D)
